# Initial kernel scaffold; baseline (speedup 1.0000x reference)
import jax
import jax.numpy as jnp
from jax.experimental import pallas as pl


def kernel(xyz, lc0, lc1, lc2, lc3, nb0, nb1, nb2, nb3, idx0, idx1, idx2, idx3, par0, par1, par2, W0, b0, W02, b02, W1, b1, W12, b12, W2, b2):
    raise NotImplementedError("write your pallas kernel here")


# speedup vs baseline: 63.7513x; 63.7513x over previous
import functools

import jax
import jax.numpy as jnp
from jax import lax
from jax.experimental import pallas as pl
from jax.experimental.pallas import tpu as pltpu
from jax.experimental.pallas import tpu_sc as plsc

_NC = 2
_NS = 16
_NW = _NC * _NS


def _tables_tc(x, p, parc, wr, wp, bias, outc):
    ms, _ = x.shape
    out = wr.shape[1]
    r = 2048 if ms % 2048 == 0 else ms
    grid = ms // r
    has_p = p is not None

    def body(*refs):
        if has_p:
            x_ref, p_ref, par_ref, wr_ref, wp_ref, b_ref, t_ref, c_ref = refs
        else:
            x_ref, par_ref, wr_ref, b_ref, t_ref, c_ref = refs
        xv = x_ref[...]
        xw = jnp.dot(xv, wr_ref[...], preferred_element_type=jnp.float32)
        t = xw
        if has_p:
            t = t + jnp.dot(p_ref[...], wp_ref[...],
                            preferred_element_type=jnp.float32)
        ep = jnp.exp(par_ref[...])
        zt = jnp.zeros((r, outc - out - 1), jnp.float32)
        t_ref[...] = jnp.concatenate([t, ep, zt], axis=1)
        zc = jnp.zeros((r, outc - out - 3), jnp.float32)
        c_ref[...] = jnp.concatenate([b_ref[...] - xw, xv, zc], axis=1)

    in_specs = [pl.BlockSpec((r, 3), lambda i: (i, 0))]
    args = [x]
    if has_p:
        in_specs.append(pl.BlockSpec((r, p.shape[1]), lambda i: (i, 0)))
        args.append(p)
    in_specs.append(pl.BlockSpec((r, 1), lambda i: (i, 0)))
    args.append(parc)
    in_specs.append(pl.BlockSpec(wr.shape, lambda i: (0, 0)))
    args.append(wr)
    if has_p:
        in_specs.append(pl.BlockSpec(wp.shape, lambda i: (0, 0)))
        args.append(wp)
    in_specs.append(pl.BlockSpec((1, out), lambda i: (0, 0)))
    args.append(bias)

    return pl.pallas_call(
        body,
        grid=(grid,),
        in_specs=in_specs,
        out_specs=[pl.BlockSpec((r, outc), lambda i: (i, 0)),
                   pl.BlockSpec((r, outc), lambda i: (i, 0))],
        out_shape=[jax.ShapeDtypeStruct((ms, outc), jnp.float32),
                   jax.ShapeDtypeStruct((ms, outc), jnp.float32)],
    )(*args)


def _sc_layer(t_full, c_full, nb_flat, idx_flat, lc_flat, wlc,
              s_sz, out, outc, mc, k):
    cpt = mc // _NW
    nchunks = cpt // k
    assert nchunks % 2 == 0 or nchunks == 1
    ov = out // 16
    mesh = plsc.VectorSubcoreMesh(core_axis_name="c", subcore_axis_name="s")

    @functools.partial(
        pl.kernel, mesh=mesh,
        compiler_params=pltpu.CompilerParams(use_tc_tiling_on_sc=False),
        out_type=[jax.ShapeDtypeStruct((mc, out), jnp.float32),
                  jax.ShapeDtypeStruct((mc, outc), jnp.float32)],
        scratch_types=[
            pltpu.VMEM((cpt,), jnp.int32),
            pltpu.VMEM((cpt, outc), jnp.float32),
            pltpu.VMEM((cpt * s_sz,), jnp.int32),
            [pltpu.VMEM((k, s_sz * 3), jnp.float32)] * 2,
            [pltpu.VMEM((k * s_sz, outc), jnp.float32)] * 2,
            [pltpu.VMEM((k, out), jnp.float32)] * 2,
            pltpu.VMEM((3, out), jnp.float32),
            [pltpu.SemaphoreType.DMA] * 2,
            pltpu.SemaphoreType.DMA,
        ])
    def sck(t_hbm, c_hbm, nb_hbm, idx_hbm, lc_hbm, wlc_hbm,
            op_hbm, ox_hbm, idx_v, cg_v, nb_v, lc_v, rows_v, op_v,
            wlc_v, sems, sem0):
        wid = lax.axis_index("s") * _NC + lax.axis_index("c")
        tbase = wid * cpt
        pltpu.sync_copy(nb_hbm.at[pl.ds(tbase * s_sz, cpt * s_sz)], nb_v)
        pltpu.sync_copy(idx_hbm.at[pl.ds(tbase, cpt)], idx_v)
        pltpu.sync_copy(wlc_hbm, wlc_v)
        pltpu.async_copy(c_hbm.at[idx_v], cg_v, sem0).wait()
        pltpu.sync_copy(cg_v, ox_hbm.at[pl.ds(tbase, cpt)])

        def issue(ci, slot):
            cb = ci * k
            pltpu.async_copy(t_hbm.at[nb_v.at[pl.ds(cb * s_sz, k * s_sz)]],
                             rows_v[slot], sems[slot])
            pltpu.async_copy(lc_hbm.at[pl.ds(tbase + cb, k)],
                             lc_v[slot], sems[slot])

        def drain(slot):
            pltpu.make_async_copy(
                t_hbm.at[pl.ds(0, k * s_sz)], rows_v[slot],
                sems[slot]).wait()
            pltpu.make_async_copy(
                lc_hbm.at[pl.ds(0, k)], lc_v[slot], sems[slot]).wait()

        def compute(ci, slot):
            cb = ci * k
            rv = rows_v[slot]
            lv = lc_v[slot]
            opv = op_v[slot]

            def center(j, carry2):
                r0 = j * s_sz
                inv = None
                for og in range(0, ov, 8):
                    ogn = min(8, ov - og)
                    wl = [[wlc_v[c, pl.ds((og + o) * 16, 16)]
                           for c in range(3)] for o in range(ogn)]
                    accs = [jnp.zeros((16,), jnp.float32)
                            for _ in range(ogn)]
                    esum = jnp.zeros((16,), jnp.float32)
                    for s in range(s_sz):
                        ew = rv[r0 + s, pl.ds(out, 16)]
                        ws = ew[0]
                        if og == 0:
                            esum = esum + ew
                        off = min(s * 3, s_sz * 3 - 16)
                        lcw = lv[j, pl.ds(off, 16)]
                        l0 = lcw[s * 3 - off]
                        l1 = lcw[s * 3 - off + 1]
                        l2 = lcw[s * 3 - off + 2]
                        for o in range(ogn):
                            v = cg_v[cb + j, pl.ds((og + o) * 16, 16)]
                            v = v + l0 * wl[o][0]
                            v = v + l1 * wl[o][1]
                            v = v + l2 * wl[o][2]
                            row = rv[r0 + s, pl.ds((og + o) * 16, 16)]
                            h = jnp.maximum(v + row, 0.0)
                            accs[o] = accs[o] + ws * h
                    if og == 0:
                        inv = 1.0 / jnp.broadcast_to(esum[0], (16,))
                    for o in range(ogn):
                        opv[j, pl.ds((og + o) * 16, 16)] = accs[o] * inv
                return carry2

            lax.fori_loop(0, k, center, 0)
            pltpu.sync_copy(opv, op_hbm.at[pl.ds(tbase + cb, k)])

        if nchunks == 1:
            issue(0, 0)
            drain(0)
            compute(0, 0)
        else:
            issue(0, 0)

            def pair(ip, carry):
                c0 = ip * 2
                drain(0)
                issue(c0 + 1, 1)
                compute(c0, 0)
                drain(1)

                @pl.when(c0 + 2 < nchunks)
                def _():
                    issue(c0 + 2, 0)

                compute(c0 + 1, 1)
                return carry

            lax.fori_loop(0, nchunks // 2, pair, 0)

    return sck(t_full, c_full, nb_flat, idx_flat, lc_flat, wlc)


def _layer(x_flat, p_flat, parc_flat, nb, idx, lc, w, b, nsrc, k):
    b_sz, npoint, s_sz = nb.shape
    out = w.shape[1]
    outc = out + 16
    wlc, wr = w[0:3], w[3:6]
    wp = w[6:] if w.shape[0] > 6 else None
    offs = jnp.arange(b_sz, dtype=jnp.int32) * nsrc
    nb_flat = (nb.astype(jnp.int32) + offs[:, None, None]).reshape(-1)
    idx_flat = (idx.astype(jnp.int32) + offs[:, None]).reshape(-1)
    lc_flat = lc.reshape(b_sz * npoint, s_sz * 3)
    t_full, c_full = _tables_tc(x_flat, p_flat, parc_flat, wr, wp,
                                b.reshape(1, out), outc)
    op, ox = _sc_layer(t_full, c_full, nb_flat, idx_flat,
                       lc_flat, wlc, s_sz, out, outc, b_sz * npoint, k)
    return ox[:, out:out + 3], op


def kernel(xyz, lc0, lc1, lc2, lc3, nb0, nb1, nb2, nb3, idx0, idx1, idx2,
           idx3, par0, par1, par2, W0, b0, W02, b02, W1, b1, W12, b12, W2,
           b2):
    b_sz, n, _ = xyz.shape
    parc0 = par0[..., 0:1].reshape(-1, 1)
    parc1 = par1[..., 0:1].reshape(-1, 1)
    parc2 = par2[..., 0:1].reshape(-1, 1)
    x = xyz.reshape(b_sz * n, 3)
    x, p = _layer(x, None, parc0, nb0, idx0, lc0, W0, b0, n, 16)
    x, p = _layer(x, p, parc0, nb0, idx0, lc0, W02, b02, n, 16)
    x, p = _layer(x, p, parc0, nb1, idx1, lc1, W1, b1, n, 8)
    x, p = _layer(x, p, parc1, nb2, idx2, lc2, W12, b12, 512, 8)
    x, p = _layer(x, p, parc2, nb3, idx3, lc3, W2, b2, 512, 4)
    npf = idx3.shape[1]
    return (x.reshape(b_sz, npf, 3), p.reshape(b_sz, npf, W2.shape[1]))


# speedup vs baseline: 67.0902x; 1.0524x over previous
import functools

import jax
import jax.numpy as jnp
from jax import lax
from jax.experimental import pallas as pl
from jax.experimental.pallas import tpu as pltpu
from jax.experimental.pallas import tpu_sc as plsc

_NC = 2
_NS = 16
_NW = _NC * _NS


def _tables_tc(xall, xoff, p, parc, wr, wp, bias, tw, outc, n_q):
    ms = xall.shape[0]
    out = wr.shape[1]
    qw = out // n_q
    r = 2048 if ms % 2048 == 0 else ms
    grid = ms // r
    has_p = p is not None

    def body(*refs):
        if has_p:
            x_ref, p_ref, par_ref, wr_ref, wp_ref, b_ref, t_ref, c_ref = refs
        else:
            x_ref, par_ref, wr_ref, b_ref, t_ref, c_ref = refs
        xv = x_ref[:, xoff:xoff + 3]
        xw = jnp.dot(xv, wr_ref[...], preferred_element_type=jnp.float32)
        t = xw
        if has_p:
            t = t + jnp.dot(p_ref[...], wp_ref[...],
                            preferred_element_type=jnp.float32)
        ep = jnp.exp(par_ref[...])
        zt = jnp.zeros((r, tw - qw - 1), jnp.float32)
        if n_q == 1:
            t_ref[...] = jnp.concatenate([t, ep, zt], axis=1)
        else:
            t_ref[...] = jnp.stack(
                [jnp.concatenate([t[:, qw * q:qw * (q + 1)], ep, zt], axis=1)
                 for q in range(n_q)], axis=0)
        zc = jnp.zeros((r, outc - out - 3), jnp.float32)
        c_ref[...] = jnp.concatenate([b_ref[...] - xw, xv, zc], axis=1)

    in_specs = [pl.BlockSpec((r, xall.shape[1]), lambda i: (i, 0))]
    args = [xall]
    if has_p:
        in_specs.append(pl.BlockSpec((r, p.shape[1]), lambda i: (i, 0)))
        args.append(p)
    in_specs.append(pl.BlockSpec((r, 1), lambda i: (i, 0)))
    args.append(parc)
    in_specs.append(pl.BlockSpec(wr.shape, lambda i: (0, 0)))
    args.append(wr)
    if has_p:
        in_specs.append(pl.BlockSpec(wp.shape, lambda i: (0, 0)))
        args.append(wp)
    in_specs.append(pl.BlockSpec((1, out), lambda i: (0, 0)))
    args.append(bias)

    if n_q == 1:
        t_spec = pl.BlockSpec((r, tw), lambda i: (i, 0))
        t_shape = jax.ShapeDtypeStruct((ms, tw), jnp.float32)
    else:
        t_spec = pl.BlockSpec((n_q, r, tw), lambda i: (0, i, 0))
        t_shape = jax.ShapeDtypeStruct((n_q, ms, tw), jnp.float32)

    return pl.pallas_call(
        body,
        grid=(grid,),
        in_specs=in_specs,
        out_specs=[t_spec, pl.BlockSpec((r, outc), lambda i: (i, 0))],
        out_shape=[t_shape,
                   jax.ShapeDtypeStruct((ms, outc), jnp.float32)],
    )(*args)


def _sc_resident(t_tab, c_full, nb_raw, idx_flat, lc_flat, wlcq,
                 s_sz, out, outc, mc, npoint, nsrc, k, n_q):
    qw = out // n_q
    tw = qw + 16
    cpt = mc // _NW if n_q == 1 else npoint
    nchunks = cpt // k
    qov = qw // 16
    nsv = s_sz // 16
    mesh = plsc.VectorSubcoreMesh(core_axis_name="c", subcore_axis_name="s")

    out_type = [jax.ShapeDtypeStruct((mc, outc), jnp.float32)]
    if n_q == 1:
        out_type.append(jax.ShapeDtypeStruct((mc, out), jnp.float32))
    else:
        out_type.append(jax.ShapeDtypeStruct((n_q, mc, qw), jnp.float32))

    @functools.partial(
        pl.kernel, mesh=mesh,
        compiler_params=pltpu.CompilerParams(use_tc_tiling_on_sc=False),
        out_type=out_type,
        scratch_types=[
            pltpu.VMEM((nsrc, tw), jnp.float32),
            pltpu.VMEM((cpt * s_sz,), jnp.int32),
            pltpu.VMEM((cpt,), jnp.int32),
            [pltpu.VMEM((k, s_sz * 3), jnp.float32)] * 2,
            [pltpu.VMEM((k, outc), jnp.float32)] * 2,
            [pltpu.VMEM((k, qw), jnp.float32)] * 2,
            pltpu.VMEM((3, qw), jnp.float32),
            [pltpu.SemaphoreType.DMA] * 2,
            pltpu.SemaphoreType.DMA,
        ])
    def sck(t_hbm, c_hbm, nb_hbm, idx_hbm, lc_hbm, wlc_hbm,
            ox_hbm, op_hbm, tt_v, nb_v, idx_v, lc_v, cg_v, op_v,
            wlc_v, sems, sem0):
        wid = lax.axis_index("s") * _NC + lax.axis_index("c")
        if n_q == 1:
            c0 = wid * cpt
            b = c0 // npoint
            q = None
            tcp = pltpu.async_copy(
                t_hbm.at[pl.ds(b * nsrc, nsrc)], tt_v, sem0)
            pltpu.sync_copy(wlc_hbm, wlc_v)
        else:
            b = wid // n_q
            q = wid % n_q
            c0 = b * npoint
            tcp = pltpu.async_copy(
                t_hbm.at[q, pl.ds(b * nsrc, nsrc)], tt_v, sem0)
            pltpu.sync_copy(wlc_hbm.at[q], wlc_v)
        pltpu.sync_copy(nb_hbm.at[pl.ds(c0 * s_sz, cpt * s_sz)], nb_v)
        pltpu.sync_copy(idx_hbm.at[pl.ds(c0, cpt)], idx_v)

        def issue(ci, slot):
            cb = ci * k
            pltpu.async_copy(c_hbm.at[idx_v.at[pl.ds(cb, k)]],
                             cg_v[slot], sems[slot])
            pltpu.async_copy(lc_hbm.at[pl.ds(c0 + cb, k)],
                             lc_v[slot], sems[slot])

        def drain(slot):
            pltpu.make_async_copy(
                c_hbm.at[pl.ds(0, k)], cg_v[slot], sems[slot]).wait()
            pltpu.make_async_copy(
                lc_hbm.at[pl.ds(0, k)], lc_v[slot], sems[slot]).wait()

        def compute(ci, slot):
            cb = ci * k
            lv = lc_v[slot]
            cgv = cg_v[slot]
            opv = op_v[slot]
            if n_q == 1:
                qb = 0
            else:
                qb = q * qw
            wl = [[wlc_v[c, pl.ds(o * 16, 16)] for c in range(3)]
                  for o in range(qov)]

            def center(j, carry2):
                cr0 = (cb + j) * s_sz
                nbw = [nb_v[pl.ds(cr0 + m * 16, 16)] for m in range(nsv)]
                accs = [jnp.zeros((16,), jnp.float32) for _ in range(qov)]
                esum = jnp.zeros((16,), jnp.float32)
                for s in range(s_sz):
                    nbl = nbw[s // 16][s % 16]
                    ew = tt_v[nbl, pl.ds(qw, 16)]
                    ws = ew[0]
                    esum = esum + ew
                    off = min(s * 3, s_sz * 3 - 16)
                    lcw = lv[j, pl.ds(off, 16)]
                    l0 = lcw[s * 3 - off]
                    l1 = lcw[s * 3 - off + 1]
                    l2 = lcw[s * 3 - off + 2]
                    for o in range(qov):
                        v = cgv[j, pl.ds(qb + o * 16, 16)]
                        v = v + l0 * wl[o][0]
                        v = v + l1 * wl[o][1]
                        v = v + l2 * wl[o][2]
                        row = tt_v[nbl, pl.ds(o * 16, 16)]
                        h = jnp.maximum(v + row, 0.0)
                        accs[o] = accs[o] + ws * h
                inv = 1.0 / jnp.broadcast_to(esum[0], (16,))
                for o in range(qov):
                    opv[j, pl.ds(o * 16, 16)] = accs[o] * inv
                return carry2

            lax.fori_loop(0, k, center, 0)
            if n_q == 1:
                pltpu.sync_copy(opv, op_hbm.at[pl.ds(c0 + cb, k)])
                pltpu.sync_copy(cgv, ox_hbm.at[pl.ds(c0 + cb, k)])
            else:
                pltpu.sync_copy(opv, op_hbm.at[q, pl.ds(c0 + cb, k)])

                @pl.when(q == 0)
                def _():
                    pltpu.sync_copy(cgv, ox_hbm.at[pl.ds(c0 + cb, k)])

        issue(0, 0)
        tcp.wait()

        def pair(ip, carry):
            cc0 = ip * 2
            drain(0)
            issue(cc0 + 1, 1)
            compute(cc0, 0)
            drain(1)

            @pl.when(cc0 + 2 < nchunks)
            def _():
                issue(cc0 + 2, 0)

            compute(cc0 + 1, 1)
            return carry

        lax.fori_loop(0, nchunks // 2, pair, 0)

    return sck(t_tab, c_full, nb_raw, idx_flat, lc_flat, wlcq)


def _layer(xall, xoff, p_flat, parc_flat, nb, idx, lc, w, b, nsrc, k, n_q):
    b_sz, npoint, s_sz = nb.shape
    mc = b_sz * npoint
    out = w.shape[1]
    outc = out + 16
    qw = out // n_q
    tw = qw + 16
    wlc, wr = w[0:3], w[3:6]
    wp = w[6:] if w.shape[0] > 6 else None
    if n_q == 1:
        wlcq = wlc
    else:
        wlcq = jnp.stack([wlc[:, qw * q:qw * (q + 1)] for q in range(n_q)],
                         axis=0)
    offs = jnp.arange(b_sz, dtype=jnp.int32) * nsrc
    nb_raw = nb.astype(jnp.int32).reshape(-1)
    idx_flat = (idx.astype(jnp.int32) + offs[:, None]).reshape(-1)
    lc_flat = lc.reshape(mc, s_sz * 3)
    t_tab, c_full = _tables_tc(xall, xoff, p_flat, parc_flat, wr, wp,
                               b.reshape(1, out), tw, outc, n_q)
    ox, op = _sc_resident(t_tab, c_full, nb_raw, idx_flat, lc_flat, wlcq,
                          s_sz, out, outc, mc, npoint, nsrc, k, n_q)
    if n_q != 1:
        op = jnp.transpose(op, (1, 0, 2)).reshape(mc, out)
    return ox, op, out


def kernel(xyz, lc0, lc1, lc2, lc3, nb0, nb1, nb2, nb3, idx0, idx1, idx2,
           idx3, par0, par1, par2, W0, b0, W02, b02, W1, b1, W12, b12, W2,
           b2):
    b_sz, n, _ = xyz.shape
    parc0 = par0[..., 0:1].reshape(-1, 1)
    parc1 = par1[..., 0:1].reshape(-1, 1)
    parc2 = par2[..., 0:1].reshape(-1, 1)
    x = xyz.reshape(b_sz * n, 3)
    ox, p, po = _layer(x, 0, None, parc0, nb0, idx0, lc0, W0, b0, n, 16, 1)
    ox, p, po = _layer(ox, po, p, parc0, nb0, idx0, lc0, W02, b02, n, 16, 1)
    ox, p, po = _layer(ox, po, p, parc0, nb1, idx1, lc1, W1, b1, n, 16, 4)
    ox, p, po = _layer(ox, po, p, parc1, nb2, idx2, lc2, W12, b12, 512, 16, 1)
    ox, p, po = _layer(ox, po, p, parc2, nb3, idx3, lc3, W2, b2, 512, 16, 4)
    npf = idx3.shape[1]
    l2x = ox[:, po:po + 3].reshape(b_sz, npf, 3)
    return (l2x, p.reshape(b_sz, npf, po))


# speedup vs baseline: 67.6181x; 1.0079x over previous
import functools

import jax
import jax.numpy as jnp
from jax import lax
from jax.experimental import pallas as pl
from jax.experimental.pallas import tpu as pltpu
from jax.experimental.pallas import tpu_sc as plsc

_NC = 2
_NS = 16
_NW = _NC * _NS


def _tables_tc(xall, xoff, p, parc, wr, wp, bias, tw, outc, n_q):
    ms = xall.shape[0]
    out = wr.shape[1]
    qw = out // n_q
    r = 2048 if ms % 2048 == 0 else ms
    grid = ms // r
    has_p = p is not None

    p_quartered = has_p and p.ndim == 3

    def body(*refs):
        if p_quartered:
            (x_ref, p0_ref, p1_ref, p2_ref, p3_ref, par_ref, wr_ref,
             wp_ref, b_ref, t_ref, c_ref) = refs
            pv = jnp.concatenate([p0_ref[0], p1_ref[0], p2_ref[0],
                                  p3_ref[0]], axis=1)
        elif has_p:
            x_ref, p_ref, par_ref, wr_ref, wp_ref, b_ref, t_ref, c_ref = refs
            pv = p_ref[...]
        else:
            x_ref, par_ref, wr_ref, b_ref, t_ref, c_ref = refs
        xv = x_ref[:, xoff:xoff + 3]
        xw = jnp.dot(xv, wr_ref[...], preferred_element_type=jnp.float32)
        t = xw
        if has_p:
            t = t + jnp.dot(pv, wp_ref[...],
                            preferred_element_type=jnp.float32)
        ep = jnp.exp(par_ref[...])
        zt = jnp.zeros((r, tw - qw - 1), jnp.float32)
        if n_q == 1:
            t_ref[...] = jnp.concatenate([t, ep, zt], axis=1)
        else:
            t_ref[...] = jnp.stack(
                [jnp.concatenate([t[:, qw * q:qw * (q + 1)], ep, zt], axis=1)
                 for q in range(n_q)], axis=0)
        zc = jnp.zeros((r, outc - out - 3), jnp.float32)
        c_ref[...] = jnp.concatenate([b_ref[...] - xw, xv, zc], axis=1)

    in_specs = [pl.BlockSpec((r, xall.shape[1]), lambda i: (i, 0))]
    args = [xall]
    if p_quartered:
        for qi in range(4):
            in_specs.append(pl.BlockSpec(
                (1, r, p.shape[2]),
                functools.partial(lambda qq, i: (qq, i, 0), qi)))
            args.append(p)
    elif has_p:
        in_specs.append(pl.BlockSpec((r, p.shape[1]), lambda i: (i, 0)))
        args.append(p)
    in_specs.append(pl.BlockSpec((r, 1), lambda i: (i, 0)))
    args.append(parc)
    in_specs.append(pl.BlockSpec(wr.shape, lambda i: (0, 0)))
    args.append(wr)
    if has_p:
        in_specs.append(pl.BlockSpec(wp.shape, lambda i: (0, 0)))
        args.append(wp)
    in_specs.append(pl.BlockSpec((1, out), lambda i: (0, 0)))
    args.append(bias)

    if n_q == 1:
        t_spec = pl.BlockSpec((r, tw), lambda i: (i, 0))
        t_shape = jax.ShapeDtypeStruct((ms, tw), jnp.float32)
    else:
        t_spec = pl.BlockSpec((n_q, r, tw), lambda i: (0, i, 0))
        t_shape = jax.ShapeDtypeStruct((n_q, ms, tw), jnp.float32)

    return pl.pallas_call(
        body,
        grid=(grid,),
        in_specs=in_specs,
        out_specs=[t_spec, pl.BlockSpec((r, outc), lambda i: (i, 0))],
        out_shape=[t_shape,
                   jax.ShapeDtypeStruct((ms, outc), jnp.float32)],
    )(*args)


def _sc_resident(t_tab, c_full, nb_raw, idx_flat, lc_flat, wlcq,
                 s_sz, out, outc, mc, npoint, nsrc, k, n_q):
    qw = out // n_q
    tw = qw + 16
    cpt = mc // _NW if n_q == 1 else npoint
    nchunks = cpt // k
    qov = qw // 16
    nsv = s_sz // 16
    mesh = plsc.VectorSubcoreMesh(core_axis_name="c", subcore_axis_name="s")

    out_type = [jax.ShapeDtypeStruct((mc, outc), jnp.float32)]
    if n_q == 1:
        out_type.append(jax.ShapeDtypeStruct((mc, out), jnp.float32))
    else:
        out_type.append(jax.ShapeDtypeStruct((n_q, mc, qw), jnp.float32))

    @functools.partial(
        pl.kernel, mesh=mesh,
        compiler_params=pltpu.CompilerParams(use_tc_tiling_on_sc=False),
        out_type=out_type,
        scratch_types=[
            pltpu.VMEM((nsrc, tw), jnp.float32),
            pltpu.VMEM((cpt, s_sz), jnp.int32),
            pltpu.VMEM((cpt,), jnp.int32),
            [pltpu.VMEM((k, s_sz * 3), jnp.float32)] * 2,
            [pltpu.VMEM((k, outc), jnp.float32)] * 2,
            [pltpu.VMEM((k, qw), jnp.float32)] * 2,
            pltpu.VMEM((3, qw), jnp.float32),
            [pltpu.SemaphoreType.DMA] * 2,
            pltpu.SemaphoreType.DMA,
        ])
    def sck(t_hbm, c_hbm, nb_hbm, idx_hbm, lc_hbm, wlc_hbm,
            ox_hbm, op_hbm, tt_v, nb_v, idx_v, lc_v, cg_v, op_v,
            wlc_v, sems, sem0):
        wid = lax.axis_index("s") * _NC + lax.axis_index("c")
        if n_q == 1:
            c0 = wid * cpt
            b = c0 // npoint
            q = None
            tcp = pltpu.async_copy(
                t_hbm.at[pl.ds(b * nsrc, nsrc)], tt_v, sem0)
            pltpu.sync_copy(wlc_hbm, wlc_v)
        else:
            b = wid // n_q
            q = wid % n_q
            c0 = b * npoint
            tcp = pltpu.async_copy(
                t_hbm.at[q, pl.ds(b * nsrc, nsrc)], tt_v, sem0)
            pltpu.sync_copy(wlc_hbm.at[q], wlc_v)
        pltpu.sync_copy(nb_hbm.at[pl.ds(c0, cpt)], nb_v)
        pltpu.sync_copy(idx_hbm.at[pl.ds(c0, cpt)], idx_v)

        def issue(ci, slot):
            cb = ci * k
            pltpu.async_copy(c_hbm.at[idx_v.at[pl.ds(cb, k)]],
                             cg_v[slot], sems[slot])
            pltpu.async_copy(lc_hbm.at[pl.ds(c0 + cb, k)],
                             lc_v[slot], sems[slot])

        def drain(slot):
            pltpu.make_async_copy(
                c_hbm.at[pl.ds(0, k)], cg_v[slot], sems[slot]).wait()
            pltpu.make_async_copy(
                lc_hbm.at[pl.ds(0, k)], lc_v[slot], sems[slot]).wait()

        def compute(ci, slot):
            cb = ci * k
            lv = lc_v[slot]
            cgv = cg_v[slot]
            opv = op_v[slot]
            if n_q == 1:
                qb = 0
            else:
                qb = q * qw
            wl = [[wlc_v[c, pl.ds(o * 16, 16)] for c in range(3)]
                  for o in range(qov)]

            def center(j, carry2):
                nbw = [nb_v[cb + j, pl.ds(m * 16, 16)] for m in range(nsv)]
                accs = [jnp.zeros((16,), jnp.float32) for _ in range(qov)]
                esum = jnp.zeros((16,), jnp.float32)
                for s in range(s_sz):
                    nbl = nbw[s // 16][s % 16]
                    ew = tt_v[nbl, pl.ds(qw, 16)]
                    ws = ew[0]
                    esum = esum + ew
                    off = min(s * 3, s_sz * 3 - 16)
                    lcw = lv[j, pl.ds(off, 16)]
                    l0 = lcw[s * 3 - off]
                    l1 = lcw[s * 3 - off + 1]
                    l2 = lcw[s * 3 - off + 2]
                    for o in range(qov):
                        v = cgv[j, pl.ds(qb + o * 16, 16)]
                        v = v + l0 * wl[o][0]
                        v = v + l1 * wl[o][1]
                        v = v + l2 * wl[o][2]
                        row = tt_v[nbl, pl.ds(o * 16, 16)]
                        h = jnp.maximum(v + row, 0.0)
                        accs[o] = accs[o] + ws * h
                inv = 1.0 / jnp.broadcast_to(esum[0], (16,))
                for o in range(qov):
                    opv[j, pl.ds(o * 16, 16)] = accs[o] * inv
                return carry2

            lax.fori_loop(0, k, center, 0)
            if n_q == 1:
                pltpu.sync_copy(opv, op_hbm.at[pl.ds(c0 + cb, k)])
                pltpu.sync_copy(cgv, ox_hbm.at[pl.ds(c0 + cb, k)])
            else:
                pltpu.sync_copy(opv, op_hbm.at[q, pl.ds(c0 + cb, k)])

                @pl.when(q == 0)
                def _():
                    pltpu.sync_copy(cgv, ox_hbm.at[pl.ds(c0 + cb, k)])

        issue(0, 0)
        tcp.wait()

        def pair(ip, carry):
            cc0 = ip * 2
            drain(0)
            issue(cc0 + 1, 1)
            compute(cc0, 0)
            drain(1)

            @pl.when(cc0 + 2 < nchunks)
            def _():
                issue(cc0 + 2, 0)

            compute(cc0 + 1, 1)
            return carry

        lax.fori_loop(0, nchunks // 2, pair, 0)

    return sck(t_tab, c_full, nb_raw, idx_flat, lc_flat, wlcq)


def _layer(xall, xoff, p_flat, parc_flat, nb, idx, lc, w, b, nsrc, k, n_q):
    b_sz, npoint, s_sz = nb.shape
    mc = b_sz * npoint
    out = w.shape[1]
    outc = out + 16
    qw = out // n_q
    tw = qw + 16
    wlc, wr = w[0:3], w[3:6]
    wp = w[6:] if w.shape[0] > 6 else None
    if n_q == 1:
        wlcq = wlc
    else:
        wlcq = jnp.stack([wlc[:, qw * q:qw * (q + 1)] for q in range(n_q)],
                         axis=0)
    offs = jnp.arange(b_sz, dtype=jnp.int32) * nsrc
    nb_raw = nb.astype(jnp.int32).reshape(mc, s_sz)
    idx_flat = (idx.astype(jnp.int32) + offs[:, None]).reshape(-1)
    lc_flat = lc.reshape(mc, s_sz * 3)
    t_tab, c_full = _tables_tc(xall, xoff, p_flat, parc_flat, wr, wp,
                               b.reshape(1, out), tw, outc, n_q)
    ox, op = _sc_resident(t_tab, c_full, nb_raw, idx_flat, lc_flat, wlcq,
                          s_sz, out, outc, mc, npoint, nsrc, k, n_q)
    return ox, op, out


def kernel(xyz, lc0, lc1, lc2, lc3, nb0, nb1, nb2, nb3, idx0, idx1, idx2,
           idx3, par0, par1, par2, W0, b0, W02, b02, W1, b1, W12, b12, W2,
           b2):
    b_sz, n, _ = xyz.shape
    parc0 = par0[..., 0:1].reshape(-1, 1)
    parc1 = par1[..., 0:1].reshape(-1, 1)
    parc2 = par2[..., 0:1].reshape(-1, 1)
    x = xyz.reshape(b_sz * n, 3)
    ox, p, po = _layer(x, 0, None, parc0, nb0, idx0, lc0, W0, b0, n, 16, 1)
    ox, p, po = _layer(ox, po, p, parc0, nb0, idx0, lc0, W02, b02, n, 16, 1)
    ox, p, po = _layer(ox, po, p, parc0, nb1, idx1, lc1, W1, b1, n, 16, 4)
    ox, p, po = _layer(ox, po, p, parc1, nb2, idx2, lc2, W12, b12, 512, 16, 1)
    ox, p, po = _layer(ox, po, p, parc2, nb3, idx3, lc3, W2, b2, 512, 16, 4)
    npf = idx3.shape[1]
    l2x = ox[:, po:po + 3].reshape(b_sz, npf, 3)
    if p.ndim == 3:
        p = jnp.transpose(p, (1, 0, 2)).reshape(b_sz * npf, po)
    return (l2x, p.reshape(b_sz, npf, po))


# speedup vs baseline: 68.2277x; 1.0090x over previous
import functools

import jax
import jax.numpy as jnp
from jax import lax
from jax.experimental import pallas as pl
from jax.experimental.pallas import tpu as pltpu
from jax.experimental.pallas import tpu_sc as plsc

_NC = 2
_NS = 16
_NW = _NC * _NS


def _tables_tc(xall, xoff, p, parc, wr, wp, bias, tw, outc, n_q):
    ms = xall.shape[0]
    out = wr.shape[1]
    qw = out // n_q
    r = 2048 if ms % 2048 == 0 else ms
    grid = ms // r
    has_p = p is not None

    p_quartered = has_p and p.ndim == 3

    def body(*refs):
        if p_quartered:
            (x_ref, p0_ref, p1_ref, p2_ref, p3_ref, par_ref, wr_ref,
             wp_ref, b_ref, t_ref, c_ref) = refs
            pv = jnp.concatenate([p0_ref[0], p1_ref[0], p2_ref[0],
                                  p3_ref[0]], axis=1)
        elif has_p:
            x_ref, p_ref, par_ref, wr_ref, wp_ref, b_ref, t_ref, c_ref = refs
            pv = p_ref[...]
        else:
            x_ref, par_ref, wr_ref, b_ref, t_ref, c_ref = refs
        xv = x_ref[:, xoff:xoff + 3]
        xw = jnp.dot(xv, wr_ref[...], preferred_element_type=jnp.float32)
        t = xw
        if has_p:
            t = t + jnp.dot(pv, wp_ref[...],
                            preferred_element_type=jnp.float32)
        ep = jnp.exp(par_ref[...])
        zt = jnp.zeros((r, tw - qw - 1), jnp.float32)
        if n_q == 1:
            t_ref[...] = jnp.concatenate([t, ep, zt], axis=1)
        else:
            t_ref[...] = jnp.stack(
                [jnp.concatenate([t[:, qw * q:qw * (q + 1)], ep, zt], axis=1)
                 for q in range(n_q)], axis=0)
        zc = jnp.zeros((r, outc - out - 3), jnp.float32)
        c_ref[...] = jnp.concatenate([b_ref[...] - xw, xv, zc], axis=1)

    in_specs = [pl.BlockSpec((r, xall.shape[1]), lambda i: (i, 0))]
    args = [xall]
    if p_quartered:
        for qi in range(4):
            in_specs.append(pl.BlockSpec(
                (1, r, p.shape[2]),
                functools.partial(lambda qq, i: (qq, i, 0), qi)))
            args.append(p)
    elif has_p:
        in_specs.append(pl.BlockSpec((r, p.shape[1]), lambda i: (i, 0)))
        args.append(p)
    in_specs.append(pl.BlockSpec((r, 1), lambda i: (i, 0)))
    args.append(parc)
    in_specs.append(pl.BlockSpec(wr.shape, lambda i: (0, 0)))
    args.append(wr)
    if has_p:
        in_specs.append(pl.BlockSpec(wp.shape, lambda i: (0, 0)))
        args.append(wp)
    in_specs.append(pl.BlockSpec((1, out), lambda i: (0, 0)))
    args.append(bias)

    if n_q == 1:
        t_spec = pl.BlockSpec((r, tw), lambda i: (i, 0))
        t_shape = jax.ShapeDtypeStruct((ms, tw), jnp.float32)
    else:
        t_spec = pl.BlockSpec((n_q, r, tw), lambda i: (0, i, 0))
        t_shape = jax.ShapeDtypeStruct((n_q, ms, tw), jnp.float32)

    return pl.pallas_call(
        body,
        grid=(grid,),
        in_specs=in_specs,
        out_specs=[t_spec, pl.BlockSpec((r, outc), lambda i: (i, 0))],
        out_shape=[t_shape,
                   jax.ShapeDtypeStruct((ms, outc), jnp.float32)],
    )(*args)


def _sc_resident(t_tab, c_full, nb_raw, idx_flat, lc_flat, wlcq,
                 s_sz, out, outc, mc, npoint, nsrc, k, n_q):
    qw = out // n_q
    tw = qw + 16
    cpt = mc // _NW if n_q == 1 else npoint
    nchunks = cpt // k
    qov = qw // 16
    nsv = s_sz // 16
    mesh = plsc.VectorSubcoreMesh(core_axis_name="c", subcore_axis_name="s")

    out_type = [jax.ShapeDtypeStruct((mc, outc), jnp.float32)]
    if n_q == 1:
        out_type.append(jax.ShapeDtypeStruct((mc, out), jnp.float32))
    else:
        out_type.append(jax.ShapeDtypeStruct((n_q, mc, qw), jnp.float32))

    @functools.partial(
        pl.kernel, mesh=mesh,
        compiler_params=pltpu.CompilerParams(use_tc_tiling_on_sc=False),
        out_type=out_type,
        scratch_types=[
            pltpu.VMEM((nsrc, tw), jnp.float32),
            pltpu.VMEM((cpt, s_sz), jnp.int32),
            pltpu.VMEM((cpt,), jnp.int32),
            [pltpu.VMEM((k, s_sz * 3), jnp.float32)] * 2,
            [pltpu.VMEM((k, outc), jnp.float32)] * 2,
            [pltpu.VMEM((k, qw), jnp.float32)] * 2,
            pltpu.VMEM((3, qw), jnp.float32),
            [pltpu.SemaphoreType.DMA] * 2,
            [pltpu.SemaphoreType.DMA] * 2,
            pltpu.SemaphoreType.DMA,
        ])
    def sck(t_hbm, c_hbm, nb_hbm, idx_hbm, lc_hbm, wlc_hbm,
            ox_hbm, op_hbm, tt_v, nb_v, idx_v, lc_v, cg_v, op_v,
            wlc_v, sems, wsems, sem0):
        wid = lax.axis_index("s") * _NC + lax.axis_index("c")
        if n_q == 1:
            c0 = wid * cpt
            b = c0 // npoint
            q = None
            tcp = pltpu.async_copy(
                t_hbm.at[pl.ds(b * nsrc, nsrc)], tt_v, sem0)
            pltpu.sync_copy(wlc_hbm, wlc_v)
        else:
            b = wid // n_q
            q = wid % n_q
            c0 = b * npoint
            tcp = pltpu.async_copy(
                t_hbm.at[q, pl.ds(b * nsrc, nsrc)], tt_v, sem0)
            pltpu.sync_copy(wlc_hbm.at[q], wlc_v)
        pltpu.sync_copy(nb_hbm.at[pl.ds(c0, cpt)], nb_v)
        pltpu.sync_copy(idx_hbm.at[pl.ds(c0, cpt)], idx_v)

        def issue(ci, slot):
            cb = ci * k
            pltpu.async_copy(c_hbm.at[idx_v.at[pl.ds(cb, k)]],
                             cg_v[slot], sems[slot])
            pltpu.async_copy(lc_hbm.at[pl.ds(c0 + cb, k)],
                             lc_v[slot], sems[slot])

        def drain(slot):
            pltpu.make_async_copy(
                c_hbm.at[pl.ds(0, k)], cg_v[slot], sems[slot]).wait()
            pltpu.make_async_copy(
                lc_hbm.at[pl.ds(0, k)], lc_v[slot], sems[slot]).wait()

        def compute(ci, slot):
            cb = ci * k
            lv = lc_v[slot]
            cgv = cg_v[slot]
            opv = op_v[slot]
            if n_q == 1:
                qb = 0
            else:
                qb = q * qw
            wl = [[wlc_v[c, pl.ds(o * 16, 16)] for c in range(3)]
                  for o in range(qov)]

            def center(j, carry2):
                nbw = [nb_v[cb + j, pl.ds(m * 16, 16)] for m in range(nsv)]
                accs = [jnp.zeros((16,), jnp.float32) for _ in range(qov)]
                esum = jnp.zeros((16,), jnp.float32)
                for s in range(s_sz):
                    nbl = nbw[s // 16][s % 16]
                    ew = tt_v[nbl, pl.ds(qw, 16)]
                    ws = ew[0]
                    esum = esum + ew
                    off = min(s * 3, s_sz * 3 - 16)
                    lcw = lv[j, pl.ds(off, 16)]
                    l0 = lcw[s * 3 - off]
                    l1 = lcw[s * 3 - off + 1]
                    l2 = lcw[s * 3 - off + 2]
                    for o in range(qov):
                        v = cgv[j, pl.ds(qb + o * 16, 16)]
                        v = v + l0 * wl[o][0]
                        v = v + l1 * wl[o][1]
                        v = v + l2 * wl[o][2]
                        row = tt_v[nbl, pl.ds(o * 16, 16)]
                        h = jnp.maximum(v + row, 0.0)
                        accs[o] = accs[o] + ws * h
                inv = 1.0 / jnp.broadcast_to(esum[0], (16,))
                for o in range(qov):
                    opv[j, pl.ds(o * 16, 16)] = accs[o] * inv
                return carry2

            lax.fori_loop(0, k, center, 0)
            if n_q == 1:
                pltpu.async_copy(opv, op_hbm.at[pl.ds(c0 + cb, k)],
                                 wsems[slot])
                pltpu.async_copy(cgv, ox_hbm.at[pl.ds(c0 + cb, k)],
                                 wsems[slot])
            else:
                pltpu.async_copy(opv, op_hbm.at[q, pl.ds(c0 + cb, k)],
                                 wsems[slot])

                @pl.when(q == 0)
                def _():
                    pltpu.async_copy(cgv, ox_hbm.at[pl.ds(c0 + cb, k)],
                                     wsems[slot])

        def drain_out(slot):
            if n_q == 1:
                pltpu.make_async_copy(
                    op_v[slot], op_hbm.at[pl.ds(0, k)], wsems[slot]).wait()
                pltpu.make_async_copy(
                    cg_v[slot], ox_hbm.at[pl.ds(0, k)], wsems[slot]).wait()
            else:
                pltpu.make_async_copy(
                    op_v[slot], op_hbm.at[0, pl.ds(0, k)],
                    wsems[slot]).wait()

                @pl.when(q == 0)
                def _():
                    pltpu.make_async_copy(
                        cg_v[slot], ox_hbm.at[pl.ds(0, k)],
                        wsems[slot]).wait()

        issue(0, 0)
        tcp.wait()

        def pair(ip, carry):
            cc0 = ip * 2
            drain(0)

            @pl.when(ip > 0)
            def _():
                drain_out(1)

            issue(cc0 + 1, 1)
            compute(cc0, 0)
            drain(1)

            @pl.when(cc0 + 2 < nchunks)
            def _():
                drain_out(0)
                issue(cc0 + 2, 0)

            compute(cc0 + 1, 1)
            return carry

        lax.fori_loop(0, nchunks // 2, pair, 0)
        drain_out(0)
        drain_out(1)

    return sck(t_tab, c_full, nb_raw, idx_flat, lc_flat, wlcq)


def _layer(xall, xoff, p_flat, parc_flat, nb, idx, lc, w, b, nsrc, k, n_q):
    b_sz, npoint, s_sz = nb.shape
    mc = b_sz * npoint
    out = w.shape[1]
    outc = out + 16
    qw = out // n_q
    tw = qw + 16
    wlc, wr = w[0:3], w[3:6]
    wp = w[6:] if w.shape[0] > 6 else None
    if n_q == 1:
        wlcq = wlc
    else:
        wlcq = jnp.stack([wlc[:, qw * q:qw * (q + 1)] for q in range(n_q)],
                         axis=0)
    offs = jnp.arange(b_sz, dtype=jnp.int32) * nsrc
    nb_raw = nb.astype(jnp.int32).reshape(mc, s_sz)
    idx_flat = (idx.astype(jnp.int32) + offs[:, None]).reshape(-1)
    lc_flat = lc.reshape(mc, s_sz * 3)
    t_tab, c_full = _tables_tc(xall, xoff, p_flat, parc_flat, wr, wp,
                               b.reshape(1, out), tw, outc, n_q)
    ox, op = _sc_resident(t_tab, c_full, nb_raw, idx_flat, lc_flat, wlcq,
                          s_sz, out, outc, mc, npoint, nsrc, k, n_q)
    return ox, op, out


def kernel(xyz, lc0, lc1, lc2, lc3, nb0, nb1, nb2, nb3, idx0, idx1, idx2,
           idx3, par0, par1, par2, W0, b0, W02, b02, W1, b1, W12, b12, W2,
           b2):
    b_sz, n, _ = xyz.shape
    parc0 = par0[..., 0:1].reshape(-1, 1)
    parc1 = par1[..., 0:1].reshape(-1, 1)
    parc2 = par2[..., 0:1].reshape(-1, 1)
    x = xyz.reshape(b_sz * n, 3)
    ox, p, po = _layer(x, 0, None, parc0, nb0, idx0, lc0, W0, b0, n, 16, 1)
    ox, p, po = _layer(ox, po, p, parc0, nb0, idx0, lc0, W02, b02, n, 16, 1)
    ox, p, po = _layer(ox, po, p, parc0, nb1, idx1, lc1, W1, b1, n, 16, 4)
    ox, p, po = _layer(ox, po, p, parc1, nb2, idx2, lc2, W12, b12, 512, 16, 1)
    ox, p, po = _layer(ox, po, p, parc2, nb3, idx3, lc3, W2, b2, 512, 16, 4)
    npf = idx3.shape[1]
    l2x = ox[:, po:po + 3].reshape(b_sz, npf, 3)
    if p.ndim == 3:
        p = jnp.transpose(p, (1, 0, 2)).reshape(b_sz * npf, po)
    return (l2x, p.reshape(b_sz, npf, po))


# speedup vs baseline: 68.6072x; 1.0056x over previous
import functools

import jax
import jax.numpy as jnp
from jax import lax
from jax.experimental import pallas as pl
from jax.experimental.pallas import tpu as pltpu
from jax.experimental.pallas import tpu_sc as plsc

_NC = 2
_NS = 16
_NW = _NC * _NS


def _tables_tc(xall, xoff, p, parc, wr, wp, bias, tw, outc, n_q):
    ms = xall.shape[0]
    out = wr.shape[1]
    qw = out // n_q
    r = 2048 if ms % 2048 == 0 else ms
    grid = ms // r
    has_p = p is not None

    p_quartered = has_p and p.ndim == 3

    def body(*refs):
        if p_quartered:
            (x_ref, p0_ref, p1_ref, p2_ref, p3_ref, par_ref, wr_ref,
             wp_ref, b_ref, t_ref, c_ref) = refs
            pv = jnp.concatenate([p0_ref[0], p1_ref[0], p2_ref[0],
                                  p3_ref[0]], axis=1)
        elif has_p:
            x_ref, p_ref, par_ref, wr_ref, wp_ref, b_ref, t_ref, c_ref = refs
            pv = p_ref[...]
        else:
            x_ref, par_ref, wr_ref, b_ref, t_ref, c_ref = refs
        xv = x_ref[:, xoff:xoff + 3]
        xw = jnp.dot(xv, wr_ref[...], preferred_element_type=jnp.float32)
        t = xw
        if has_p:
            t = t + jnp.dot(pv, wp_ref[...],
                            preferred_element_type=jnp.float32)
        ep = jnp.exp(par_ref[...])
        zt = jnp.zeros((r, tw - qw - 1), jnp.float32)
        if n_q == 1:
            t_ref[...] = jnp.concatenate([t, ep, zt], axis=1)
        else:
            t_ref[...] = jnp.stack(
                [jnp.concatenate([t[:, qw * q:qw * (q + 1)], ep, zt], axis=1)
                 for q in range(n_q)], axis=0)
        zc = jnp.zeros((r, outc - out - 3), jnp.float32)
        c_ref[...] = jnp.concatenate([b_ref[...] - xw, xv, zc], axis=1)

    in_specs = [pl.BlockSpec((r, xall.shape[1]), lambda i: (i, 0))]
    args = [xall]
    if p_quartered:
        for qi in range(4):
            in_specs.append(pl.BlockSpec(
                (1, r, p.shape[2]),
                functools.partial(lambda qq, i: (qq, i, 0), qi)))
            args.append(p)
    elif has_p:
        in_specs.append(pl.BlockSpec((r, p.shape[1]), lambda i: (i, 0)))
        args.append(p)
    in_specs.append(pl.BlockSpec((r, 1), lambda i: (i, 0)))
    args.append(parc)
    in_specs.append(pl.BlockSpec(wr.shape, lambda i: (0, 0)))
    args.append(wr)
    if has_p:
        in_specs.append(pl.BlockSpec(wp.shape, lambda i: (0, 0)))
        args.append(wp)
    in_specs.append(pl.BlockSpec((1, out), lambda i: (0, 0)))
    args.append(bias)

    if n_q == 1:
        t_spec = pl.BlockSpec((r, tw), lambda i: (i, 0))
        t_shape = jax.ShapeDtypeStruct((ms, tw), jnp.float32)
    else:
        t_spec = pl.BlockSpec((n_q, r, tw), lambda i: (0, i, 0))
        t_shape = jax.ShapeDtypeStruct((n_q, ms, tw), jnp.float32)

    return pl.pallas_call(
        body,
        grid=(grid,),
        in_specs=in_specs,
        out_specs=[t_spec, pl.BlockSpec((r, outc), lambda i: (i, 0))],
        out_shape=[t_shape,
                   jax.ShapeDtypeStruct((ms, outc), jnp.float32)],
    )(*args)


def _sc_resident(t_tab, c_full, nb_raw, idx_flat, lc_flat, wlcq,
                 s_sz, out, outc, mc, npoint, nsrc, k, n_q):
    qw = out // n_q
    tw = qw + 16
    cpt = mc // _NW if n_q == 1 else npoint
    nchunks = cpt // k
    qov = qw // 16
    nsv = s_sz // 16
    mesh = plsc.VectorSubcoreMesh(core_axis_name="c", subcore_axis_name="s")

    out_type = [jax.ShapeDtypeStruct((mc, outc), jnp.float32)]
    if n_q == 1:
        out_type.append(jax.ShapeDtypeStruct((mc, out), jnp.float32))
    else:
        out_type.append(jax.ShapeDtypeStruct((n_q, mc, qw), jnp.float32))

    @functools.partial(
        pl.kernel, mesh=mesh,
        compiler_params=pltpu.CompilerParams(use_tc_tiling_on_sc=False),
        out_type=out_type,
        scratch_types=[
            pltpu.VMEM((nsrc, tw), jnp.float32),
            pltpu.VMEM((cpt, s_sz), jnp.int32),
            pltpu.VMEM((cpt,), jnp.int32),
            [pltpu.VMEM((k, s_sz * 3), jnp.float32)] * 2,
            [pltpu.VMEM((k, outc), jnp.float32)] * 2,
            [pltpu.VMEM((k, qw), jnp.float32)] * 2,
            pltpu.VMEM((3, qw), jnp.float32),
            [pltpu.SemaphoreType.DMA] * 2,
            [pltpu.SemaphoreType.DMA] * 2,
            pltpu.SemaphoreType.DMA,
        ])
    def sck(t_hbm, c_hbm, nb_hbm, idx_hbm, lc_hbm, wlc_hbm,
            ox_hbm, op_hbm, tt_v, nb_v, idx_v, lc_v, cg_v, op_v,
            wlc_v, sems, wsems, sem0):
        wid = lax.axis_index("s") * _NC + lax.axis_index("c")
        if n_q == 1:
            c0 = wid * cpt
            b = c0 // npoint
            q = None
            tcp = pltpu.async_copy(
                t_hbm.at[pl.ds(b * nsrc, nsrc)], tt_v, sem0)
            pltpu.sync_copy(wlc_hbm, wlc_v)
        else:
            b = wid // n_q
            q = wid % n_q
            c0 = b * npoint
            tcp = pltpu.async_copy(
                t_hbm.at[q, pl.ds(b * nsrc, nsrc)], tt_v, sem0)
            pltpu.sync_copy(wlc_hbm.at[q], wlc_v)
        pltpu.sync_copy(nb_hbm.at[pl.ds(c0, cpt)], nb_v)
        pltpu.sync_copy(idx_hbm.at[pl.ds(c0, cpt)], idx_v)

        def issue(ci, slot):
            cb = ci * k
            pltpu.async_copy(c_hbm.at[idx_v.at[pl.ds(cb, k)]],
                             cg_v[slot], sems[slot])
            pltpu.async_copy(lc_hbm.at[pl.ds(c0 + cb, k)],
                             lc_v[slot], sems[slot])

        def drain(slot):
            pltpu.make_async_copy(
                c_hbm.at[pl.ds(0, k)], cg_v[slot], sems[slot]).wait()
            pltpu.make_async_copy(
                lc_hbm.at[pl.ds(0, k)], lc_v[slot], sems[slot]).wait()

        def compute(ci, slot):
            cb = ci * k
            lv = lc_v[slot]
            cgv = cg_v[slot]
            opv = op_v[slot]
            if n_q == 1:
                qb = 0
            else:
                qb = q * qw
            wl = [[wlc_v[c, pl.ds(o * 16, 16)] for c in range(3)]
                  for o in range(qov)]

            def center(j, carry2):
                nbw = [nb_v[cb + j, pl.ds(m * 16, 16)] for m in range(nsv)]
                accs = [jnp.zeros((16,), jnp.float32) for _ in range(qov)]
                esum = jnp.zeros((16,), jnp.float32)
                for s in range(s_sz):
                    nbl = nbw[s // 16][s % 16]
                    ew = tt_v[nbl, pl.ds(qw, 16)]
                    ws = ew[0]
                    esum = esum + ew
                    off = min(s * 3, s_sz * 3 - 16)
                    lcw = lv[j, pl.ds(off, 16)]
                    l0 = lcw[s * 3 - off]
                    l1 = lcw[s * 3 - off + 1]
                    l2 = lcw[s * 3 - off + 2]
                    for o in range(qov):
                        v = cgv[j, pl.ds(qb + o * 16, 16)]
                        v = v + l0 * wl[o][0]
                        v = v + l1 * wl[o][1]
                        v = v + l2 * wl[o][2]
                        row = tt_v[nbl, pl.ds(o * 16, 16)]
                        h = jnp.maximum(v + row, 0.0)
                        accs[o] = accs[o] + ws * h
                inv = 1.0 / jnp.broadcast_to(esum[0], (16,))
                for o in range(qov):
                    opv[j, pl.ds(o * 16, 16)] = accs[o] * inv
                return carry2

            lax.fori_loop(0, k, center, 0)
            if n_q == 1:
                pltpu.async_copy(opv, op_hbm.at[pl.ds(c0 + cb, k)],
                                 wsems[slot])
                pltpu.async_copy(cgv, ox_hbm.at[pl.ds(c0 + cb, k)],
                                 wsems[slot])
            else:
                pltpu.async_copy(opv, op_hbm.at[q, pl.ds(c0 + cb, k)],
                                 wsems[slot])

                @pl.when(q == 0)
                def _():
                    pltpu.async_copy(cgv, ox_hbm.at[pl.ds(c0 + cb, k)],
                                     wsems[slot])

        def drain_out(slot):
            if n_q == 1:
                pltpu.make_async_copy(
                    op_v[slot], op_hbm.at[pl.ds(0, k)], wsems[slot]).wait()
                pltpu.make_async_copy(
                    cg_v[slot], ox_hbm.at[pl.ds(0, k)], wsems[slot]).wait()
            else:
                pltpu.make_async_copy(
                    op_v[slot], op_hbm.at[0, pl.ds(0, k)],
                    wsems[slot]).wait()

                @pl.when(q == 0)
                def _():
                    pltpu.make_async_copy(
                        cg_v[slot], ox_hbm.at[pl.ds(0, k)],
                        wsems[slot]).wait()

        issue(0, 0)
        tcp.wait()

        def pair(ip, carry):
            cc0 = ip * 2
            drain(0)

            @pl.when(ip > 0)
            def _():
                drain_out(1)

            issue(cc0 + 1, 1)
            compute(cc0, 0)
            drain(1)

            @pl.when(cc0 + 2 < nchunks)
            def _():
                drain_out(0)
                issue(cc0 + 2, 0)

            compute(cc0 + 1, 1)
            return carry

        lax.fori_loop(0, nchunks // 2, pair, 0)
        drain_out(0)
        drain_out(1)

    return sck(t_tab, c_full, nb_raw, idx_flat, lc_flat, wlcq)


def _layer(xall, xoff, p_flat, parc_flat, nb, idx, lc, w, b, nsrc, k, n_q):
    b_sz, npoint, s_sz = nb.shape
    mc = b_sz * npoint
    out = w.shape[1]
    outc = out + 16
    qw = out // n_q
    tw = qw + 16
    wlc, wr = w[0:3], w[3:6]
    wp = w[6:] if w.shape[0] > 6 else None
    if n_q == 1:
        wlcq = wlc
    else:
        wlcq = jnp.stack([wlc[:, qw * q:qw * (q + 1)] for q in range(n_q)],
                         axis=0)
    offs = jnp.arange(b_sz, dtype=jnp.int32) * nsrc
    nb_raw = nb.astype(jnp.int32).reshape(mc, s_sz)
    idx_flat = (idx.astype(jnp.int32) + offs[:, None]).reshape(-1)
    lc_flat = lc.reshape(mc, s_sz * 3)
    t_tab, c_full = _tables_tc(xall, xoff, p_flat, parc_flat, wr, wp,
                               b.reshape(1, out), tw, outc, n_q)
    ox, op = _sc_resident(t_tab, c_full, nb_raw, idx_flat, lc_flat, wlcq,
                          s_sz, out, outc, mc, npoint, nsrc, k, n_q)
    return ox, op, out


def kernel(xyz, lc0, lc1, lc2, lc3, nb0, nb1, nb2, nb3, idx0, idx1, idx2,
           idx3, par0, par1, par2, W0, b0, W02, b02, W1, b1, W12, b12, W2,
           b2):
    b_sz, n, _ = xyz.shape
    parc0 = par0[..., 0:1].reshape(-1, 1)
    parc1 = par1[..., 0:1].reshape(-1, 1)
    parc2 = par2[..., 0:1].reshape(-1, 1)
    x = xyz.reshape(b_sz * n, 3)
    ox, p, po = _layer(x, 0, None, parc0, nb0, idx0, lc0, W0, b0, n, 32, 1)
    ox, p, po = _layer(ox, po, p, parc0, nb0, idx0, lc0, W02, b02, n, 32, 1)
    ox, p, po = _layer(ox, po, p, parc0, nb1, idx1, lc1, W1, b1, n, 16, 4)
    ox, p, po = _layer(ox, po, p, parc1, nb2, idx2, lc2, W12, b12, 512, 32, 1)
    ox, p, po = _layer(ox, po, p, parc2, nb3, idx3, lc3, W2, b2, 512, 32, 4)
    npf = idx3.shape[1]
    l2x = ox[:, po:po + 3].reshape(b_sz, npf, 3)
    if p.ndim == 3:
        p = jnp.transpose(p, (1, 0, 2)).reshape(b_sz * npf, po)
    return (l2x, p.reshape(b_sz, npf, po))
